# baseline (speedup 1.0000x reference)
import functools
import jax
import jax.numpy as jnp
from jax import lax
from jax.experimental import pallas as pl
from jax.experimental.pallas import tpu as pltpu, tpu_sc as plsc

ROWS = 16384
DIM = 2048
L = 16
NC = 2
NS = 16
NW = NC * NS
ROWS_PER_W = ROWS // NW
CHUNK = 16
N_CHUNKS = ROWS_PER_W // CHUNK
JBLKS = DIM // L

_mesh = plsc.VectorSubcoreMesh(core_axis_name="c", subcore_axis_name="s")


@functools.partial(
    pl.kernel,
    out_type=jax.ShapeDtypeStruct((ROWS * DIM,), jnp.float32),
    mesh=_mesh,
    compiler_params=pltpu.CompilerParams(needs_layout_passes=False),
    scratch_types=[
        pltpu.VMEM((DIM,), jnp.int32),
        pltpu.VMEM((CHUNK * DIM,), jnp.float32),
        pltpu.VMEM((CHUNK * DIM,), jnp.float32),
    ],
)
def _permute_sc(x_hbm, idx_hbm, out_hbm, idx_v, in_v, out_v):
    wid = lax.axis_index("s") * NC + lax.axis_index("c")
    w_base = wid * ROWS_PER_W

    pltpu.sync_copy(idx_hbm, idx_v)

    def chunk_body(c, carry):
        base = (w_base + c * CHUNK) * DIM
        pltpu.sync_copy(x_hbm.at[pl.ds(base, CHUNK * DIM)], in_v)

        def row_body(i, carry2):
            row_base = i * DIM
            for j in range(JBLKS):
                idx_vec = idx_v[pl.ds(j * L, L)] + row_base
                v = plsc.load_gather(in_v, [idx_vec])
                out_v[pl.ds(row_base + j * L, L)] = v
            return carry2

        lax.fori_loop(0, CHUNK, row_body, 0)
        pltpu.sync_copy(out_v, out_hbm.at[pl.ds(base, CHUNK * DIM)])
        return carry

    lax.fori_loop(0, N_CHUNKS, chunk_body, 0)


def kernel(x, indices):
    out_flat = _permute_sc(x.reshape(-1), indices)
    return out_flat.reshape(ROWS, DIM)


# speedup vs baseline: 1.8205x; 1.8205x over previous
import functools
import jax
import jax.numpy as jnp
from jax import lax
from jax.experimental import pallas as pl
from jax.experimental.pallas import tpu as pltpu, tpu_sc as plsc

ROWS = 16384
DIM = 2048
L = 16
NC = 2
NS = 16
NW = NC * NS
ROWS_PER_W = ROWS // NW
CHUNK = 8
CB = CHUNK * DIM
N_CHUNKS = ROWS_PER_W // CHUNK
JBLKS = DIM // L
GROUP = 16

_mesh = plsc.VectorSubcoreMesh(core_axis_name="c", subcore_axis_name="s")


@functools.partial(
    pl.kernel,
    out_type=jax.ShapeDtypeStruct((ROWS * DIM,), jnp.float32),
    mesh=_mesh,
    compiler_params=pltpu.CompilerParams(needs_layout_passes=False),
    scratch_types=[
        pltpu.VMEM((DIM,), jnp.int32),
        pltpu.VMEM((CB,), jnp.float32),
        pltpu.VMEM((CB,), jnp.float32),
        pltpu.VMEM((CB,), jnp.float32),
        pltpu.VMEM((CB,), jnp.float32),
        pltpu.SemaphoreType.DMA,
        pltpu.SemaphoreType.DMA,
        pltpu.SemaphoreType.DMA,
        pltpu.SemaphoreType.DMA,
    ],
)
def _permute_sc(x_hbm, idx_hbm, out_hbm, idx_v, in0, in1, out0, out1,
                in_s0, in_s1, out_s0, out_s1):
    wid = lax.axis_index("s") * NC + lax.axis_index("c")
    w_base = wid * ROWS_PER_W * DIM

    in_bufs = (in0, in1)
    out_bufs = (out0, out1)
    in_sems = (in_s0, in_s1)
    out_sems = (out_s0, out_s1)

    pltpu.sync_copy(idx_hbm, idx_v)

    def start_in(c, b):
        pltpu.async_copy(x_hbm.at[pl.ds(w_base + c * CB, CB)],
                         in_bufs[b], in_sems[b])

    def wait_in(b):
        pltpu.make_async_copy(x_hbm.at[pl.ds(0, CB)],
                              in_bufs[b], in_sems[b]).wait()

    def start_out(c, b):
        pltpu.async_copy(out_bufs[b],
                         out_hbm.at[pl.ds(w_base + c * CB, CB)], out_sems[b])

    def wait_out(b):
        pltpu.make_async_copy(out_bufs[b],
                              out_hbm.at[pl.ds(0, CB)], out_sems[b]).wait()

    def gather_chunk(b):
        in_buf = in_bufs[b]
        out_buf = out_bufs[b]
        for g in range(JBLKS // GROUP):
            idxs = [idx_v[pl.ds((g * GROUP + j) * L, L)] for j in range(GROUP)]

            def row_body(i, acc):
                rb = i * DIM
                ob = rb + g * (GROUP * L)
                for j in range(GROUP):
                    v = plsc.load_gather(in_buf, [idxs[j] + rb])
                    out_buf[pl.ds(ob + j * L, L)] = v
                return acc

            lax.fori_loop(0, CHUNK, row_body, 0)

    start_in(0, 0)
    start_in(1, 1)
    for b in range(2):
        wait_in(b)
        gather_chunk(b)
        start_in(b + 2, b)
        start_out(b, b)

    def pair_body(k, acc):
        for b in range(2):
            c = 2 + 2 * k + b
            wait_in(b)
            wait_out(b)
            gather_chunk(b)
            start_in(c + 2, b)
            start_out(c, b)
        return acc

    lax.fori_loop(0, (N_CHUNKS - 4) // 2, pair_body, 0)

    for b in range(2):
        c = N_CHUNKS - 2 + b
        wait_in(b)
        wait_out(b)
        gather_chunk(b)
        start_out(c, b)
    for b in range(2):
        wait_out(b)


def kernel(x, indices):
    out_flat = _permute_sc(x.reshape(-1), indices)
    return out_flat.reshape(ROWS, DIM)


# speedup vs baseline: 3.2930x; 1.8089x over previous
import functools
import jax
import jax.numpy as jnp
from jax import lax
from jax.experimental import pallas as pl
from jax.experimental.pallas import tpu as pltpu, tpu_sc as plsc

ROWS = 16384
DIM = 2048
L = 16
NC = 2
NS = 16
NW = NC * NS
ROWS_PER_W = ROWS // NW
CHUNK = 8
CB = CHUNK * DIM
N_CHUNKS = ROWS_PER_W // CHUNK
JBLKS = DIM // L
GROUP = 16

_mesh = plsc.VectorSubcoreMesh(core_axis_name="c", subcore_axis_name="s")


@functools.partial(
    pl.kernel,
    out_type=jax.ShapeDtypeStruct((ROWS, DIM), jnp.float32),
    mesh=_mesh,
    compiler_params=pltpu.CompilerParams(needs_layout_passes=False),
    scratch_types=[
        pltpu.VMEM((DIM,), jnp.int32),
        pltpu.VMEM((CHUNK, DIM), jnp.float32),
        pltpu.VMEM((CHUNK, DIM), jnp.float32),
        pltpu.VMEM((CHUNK, DIM), jnp.float32),
        pltpu.VMEM((CHUNK, DIM), jnp.float32),
        pltpu.SemaphoreType.DMA,
        pltpu.SemaphoreType.DMA,
        pltpu.SemaphoreType.DMA,
        pltpu.SemaphoreType.DMA,
    ],
)
def _permute_sc(x_hbm, idx_hbm, out_hbm, idx_v, in0, in1, out0, out1,
                in_s0, in_s1, out_s0, out_s1):
    wid = lax.axis_index("s") * NC + lax.axis_index("c")
    w_row = wid * ROWS_PER_W

    in_bufs = (in0, in1)
    out_bufs = (out0, out1)
    in_sems = (in_s0, in_s1)
    out_sems = (out_s0, out_s1)

    pltpu.sync_copy(idx_hbm, idx_v)

    def start_in(c, b):
        pltpu.async_copy(x_hbm.at[pl.ds(w_row + c * CHUNK, CHUNK)],
                         in_bufs[b], in_sems[b])

    def wait_in(b):
        pltpu.make_async_copy(x_hbm.at[pl.ds(0, CHUNK)],
                              in_bufs[b], in_sems[b]).wait()

    def start_out(c, b):
        pltpu.async_copy(out_bufs[b],
                         out_hbm.at[pl.ds(w_row + c * CHUNK, CHUNK)],
                         out_sems[b])

    def wait_out(b):
        pltpu.make_async_copy(out_bufs[b],
                              out_hbm.at[pl.ds(0, CHUNK)], out_sems[b]).wait()

    def gather_chunk(b):
        in_buf = in_bufs[b]
        out_buf = out_bufs[b]

        def g_body(g, acc):
            gbase = g * (GROUP * L)
            idxs = [idx_v[pl.ds(gbase + j * L, L)] for j in range(GROUP)]

            def row_body(i, acc2):
                row_splat = jnp.full((L,), i, dtype=jnp.int32)
                for j in range(GROUP):
                    v = plsc.load_gather(in_buf, [row_splat, idxs[j]])
                    out_buf[i, pl.ds(gbase + j * L, L)] = v
                return acc2

            lax.fori_loop(0, CHUNK, row_body, 0)
            return acc

        lax.fori_loop(0, JBLKS // GROUP, g_body, 0)

    start_in(0, 0)
    start_in(1, 1)
    for b in range(2):
        wait_in(b)
        gather_chunk(b)
        start_in(b + 2, b)
        start_out(b, b)

    def pair_body(k, acc):
        for b in range(2):
            c = 2 + 2 * k + b
            wait_in(b)
            wait_out(b)
            gather_chunk(b)
            start_in(c + 2, b)
            start_out(c, b)
        return acc

    lax.fori_loop(0, (N_CHUNKS - 4) // 2, pair_body, 0)

    for b in range(2):
        c = N_CHUNKS - 2 + b
        wait_in(b)
        wait_out(b)
        gather_chunk(b)
        start_out(c, b)
    for b in range(2):
        wait_out(b)


def kernel(x, indices):
    return _permute_sc(x, indices)


# speedup vs baseline: 5.1044x; 1.5501x over previous
import functools
import jax
import jax.numpy as jnp
from jax import lax
from jax.experimental import pallas as pl
from jax.experimental.pallas import tpu as pltpu, tpu_sc as plsc

ROWS = 16384
DIM = 2048
L = 16
NC = 2
NS = 16
NW = NC * NS
ROWS_PER_W = ROWS // NW
CHUNK = 8
CB = CHUNK * DIM
N_CHUNKS = ROWS_PER_W // CHUNK
JBLKS = DIM // L
GROUP = 16

_mesh = plsc.VectorSubcoreMesh(core_axis_name="c", subcore_axis_name="s")


@functools.partial(
    pl.kernel,
    out_type=jax.ShapeDtypeStruct((ROWS, DIM), jnp.float32),
    mesh=_mesh,
    compiler_params=pltpu.CompilerParams(needs_layout_passes=False),
    scratch_types=[
        pltpu.VMEM((DIM,), jnp.int32),
        pltpu.VMEM((CHUNK, DIM), jnp.float32),
        pltpu.VMEM((CHUNK, DIM), jnp.float32),
        pltpu.VMEM((CHUNK, DIM), jnp.float32),
        pltpu.VMEM((CHUNK, DIM), jnp.float32),
        pltpu.SemaphoreType.DMA,
        pltpu.SemaphoreType.DMA,
        pltpu.SemaphoreType.DMA,
        pltpu.SemaphoreType.DMA,
    ],
)
def _permute_sc(x_hbm, idx_hbm, out_hbm, idx_v, in0, in1, out0, out1,
                in_s0, in_s1, out_s0, out_s1):
    wid = lax.axis_index("s") * NC + lax.axis_index("c")
    w_row = wid * ROWS_PER_W

    in_bufs = (in0, in1)
    out_bufs = (out0, out1)
    in_sems = (in_s0, in_s1)
    out_sems = (out_s0, out_s1)

    pltpu.sync_copy(idx_hbm, idx_v)

    def start_in(c, b):
        pltpu.async_copy(x_hbm.at[pl.ds(w_row + c * CHUNK, CHUNK)],
                         in_bufs[b], in_sems[b])

    def wait_in(b):
        pltpu.make_async_copy(x_hbm.at[pl.ds(0, CHUNK)],
                              in_bufs[b], in_sems[b]).wait()

    def start_out(c, b):
        pltpu.async_copy(out_bufs[b],
                         out_hbm.at[pl.ds(w_row + c * CHUNK, CHUNK)],
                         out_sems[b])

    def wait_out(b):
        pltpu.make_async_copy(out_bufs[b],
                              out_hbm.at[pl.ds(0, CHUNK)], out_sems[b]).wait()

    def gather_chunk(b):
        in_buf = in_bufs[b]
        out_buf = out_bufs[b]
        for g in range(JBLKS // GROUP):
            gbase = g * (GROUP * L)
            idxs = [idx_v[pl.ds(gbase + j * L, L)] for j in range(GROUP)]

            @plsc.parallel_loop(0, CHUNK)
            def row_body(i):
                row_splat = jnp.full((L,), i, dtype=jnp.int32)
                for j in range(GROUP):
                    v = plsc.load_gather(in_buf, [row_splat, idxs[j]])
                    out_buf[i, pl.ds(gbase + j * L, L)] = v

    start_in(0, 0)
    start_in(1, 1)

    def pair_body(k, acc):
        for b in range(2):
            c = 2 * k + b
            wait_in(b)

            @pl.when(c >= 2)
            def _():
                wait_out(b)

            gather_chunk(b)

            @pl.when(c + 2 < N_CHUNKS)
            def _():
                start_in(c + 2, b)

            start_out(c, b)
        return acc

    lax.fori_loop(0, N_CHUNKS // 2, pair_body, 0)
    for b in range(2):
        wait_out(b)


def kernel(x, indices):
    return _permute_sc(x, indices)


# speedup vs baseline: 5.3043x; 1.0392x over previous
import functools
import jax
import jax.numpy as jnp
from jax import lax
from jax.experimental import pallas as pl
from jax.experimental.pallas import tpu as pltpu, tpu_sc as plsc

ROWS = 16384
DIM = 2048
L = 16
NC = 2
NS = 16
NW = NC * NS
ROWS_PER_W = ROWS // NW
CHUNK = 8
CB = CHUNK * DIM
N_CHUNKS = ROWS_PER_W // CHUNK
JBLKS = DIM // L
GROUP = 16

_mesh = plsc.VectorSubcoreMesh(core_axis_name="c", subcore_axis_name="s")


@functools.partial(
    pl.kernel,
    out_type=jax.ShapeDtypeStruct((ROWS, DIM), jnp.float32),
    mesh=_mesh,
    compiler_params=pltpu.CompilerParams(needs_layout_passes=False),
    scratch_types=[
        pltpu.VMEM((DIM,), jnp.int32),
        pltpu.VMEM((CB,), jnp.float32),
        pltpu.VMEM((CB,), jnp.float32),
        pltpu.VMEM((CB,), jnp.float32),
        pltpu.VMEM((CB,), jnp.float32),
        pltpu.SemaphoreType.DMA,
        pltpu.SemaphoreType.DMA,
        pltpu.SemaphoreType.DMA,
        pltpu.SemaphoreType.DMA,
    ],
)
def _permute_sc(x_hbm, idx_hbm, out_hbm, idx_v, in0, in1, out0, out1,
                in_s0, in_s1, out_s0, out_s1):
    wid = lax.axis_index("s") * NC + lax.axis_index("c")
    w_row = wid * ROWS_PER_W

    in_bufs = (in0, in1)
    out_bufs = (out0, out1)
    in_sems = (in_s0, in_s1)
    out_sems = (out_s0, out_s1)

    pltpu.sync_copy(idx_hbm, idx_v)

    def start_in(c, b):
        row = w_row + c * CHUNK
        for i in range(CHUNK):
            pltpu.async_copy(x_hbm.at[row + i],
                             in_bufs[b].at[pl.ds(i * DIM, DIM)], in_sems[b])

    def wait_in(b):
        for i in range(CHUNK):
            pltpu.make_async_copy(x_hbm.at[0],
                                  in_bufs[b].at[pl.ds(i * DIM, DIM)],
                                  in_sems[b]).wait()

    def start_out(c, b):
        row = w_row + c * CHUNK
        for i in range(CHUNK):
            pltpu.async_copy(out_bufs[b].at[pl.ds(i * DIM, DIM)],
                             out_hbm.at[row + i], out_sems[b])

    def wait_out(b):
        for i in range(CHUNK):
            pltpu.make_async_copy(out_bufs[b].at[pl.ds(i * DIM, DIM)],
                                  out_hbm.at[0], out_sems[b]).wait()

    def gather_chunk(b):
        in_buf = in_bufs[b]
        out_buf = out_bufs[b]
        for g in range(JBLKS // GROUP):
            gbase = g * (GROUP * L)
            idxs = [idx_v[pl.ds(gbase + j * L, L)] for j in range(GROUP)]

            @plsc.parallel_loop(0, CHUNK)
            def row_body(i):
                row_ref = in_buf.at[pl.ds(i * DIM, DIM)]
                obase = i * DIM + gbase
                for j in range(GROUP):
                    v = plsc.load_gather(row_ref, [idxs[j]])
                    out_buf[pl.ds(obase + j * L, L)] = v

    start_in(0, 0)
    start_in(1, 1)

    def pair_body(k, acc):
        for b in range(2):
            c = 2 * k + b
            wait_in(b)

            @pl.when(c >= 2)
            def _():
                wait_out(b)

            gather_chunk(b)

            @pl.when(c + 2 < N_CHUNKS)
            def _():
                start_in(c + 2, b)

            start_out(c, b)
        return acc

    lax.fori_loop(0, N_CHUNKS // 2, pair_body, 0)
    for b in range(2):
        wait_out(b)


def kernel(x, indices):
    return _permute_sc(x, indices)


# speedup vs baseline: 5.5687x; 1.0499x over previous
import functools
import jax
import jax.numpy as jnp
from jax import lax
from jax.experimental import pallas as pl
from jax.experimental.pallas import tpu as pltpu, tpu_sc as plsc

ROWS = 16384
DIM = 2048
L = 16
NC = 2
NS = 16
NW = NC * NS
ROWS_PER_W = ROWS // NW
CHUNK = 8
CB = CHUNK * DIM
N_CHUNKS = ROWS_PER_W // CHUNK
JBLKS = DIM // L
GROUP = 32

_mesh = plsc.VectorSubcoreMesh(core_axis_name="c", subcore_axis_name="s")


@functools.partial(
    pl.kernel,
    out_type=jax.ShapeDtypeStruct((ROWS, DIM), jnp.float32),
    mesh=_mesh,
    compiler_params=pltpu.CompilerParams(needs_layout_passes=False),
    scratch_types=[
        pltpu.VMEM((DIM,), jnp.int32),
        pltpu.VMEM((CB,), jnp.float32),
        pltpu.VMEM((CB,), jnp.float32),
        pltpu.VMEM((CB,), jnp.float32),
        pltpu.VMEM((CB,), jnp.float32),
        pltpu.SemaphoreType.DMA,
        pltpu.SemaphoreType.DMA,
        pltpu.SemaphoreType.DMA,
        pltpu.SemaphoreType.DMA,
    ],
)
def _permute_sc(x_hbm, idx_hbm, out_hbm, idx_v, in0, in1, out0, out1,
                in_s0, in_s1, out_s0, out_s1):
    wid = lax.axis_index("s") * NC + lax.axis_index("c")
    w_row = wid * ROWS_PER_W

    in_bufs = (in0, in1)
    out_bufs = (out0, out1)
    in_sems = (in_s0, in_s1)
    out_sems = (out_s0, out_s1)

    pltpu.sync_copy(idx_hbm, idx_v)

    def start_in(c, b):
        row = w_row + c * CHUNK
        for i in range(CHUNK):
            pltpu.async_copy(x_hbm.at[row + i],
                             in_bufs[b].at[pl.ds(i * DIM, DIM)], in_sems[b])

    def wait_in(b):
        for i in range(CHUNK):
            pltpu.make_async_copy(x_hbm.at[0],
                                  in_bufs[b].at[pl.ds(i * DIM, DIM)],
                                  in_sems[b]).wait()

    def start_out(c, b):
        row = w_row + c * CHUNK
        for i in range(CHUNK):
            pltpu.async_copy(out_bufs[b].at[pl.ds(i * DIM, DIM)],
                             out_hbm.at[row + i], out_sems[b])

    def wait_out(b):
        for i in range(CHUNK):
            pltpu.make_async_copy(out_bufs[b].at[pl.ds(i * DIM, DIM)],
                                  out_hbm.at[0], out_sems[b]).wait()

    def gather_chunk(b):
        in_buf = in_bufs[b]
        out_buf = out_bufs[b]
        for g in range(JBLKS // GROUP):
            gbase = g * (GROUP * L)
            idxs = [idx_v[pl.ds(gbase + j * L, L)] for j in range(GROUP)]

            @plsc.parallel_loop(0, CHUNK)
            def row_body(i):
                row_ref = in_buf.at[pl.ds(i * DIM, DIM)]
                obase = i * DIM + gbase
                for j in range(GROUP):
                    v = plsc.load_gather(row_ref, [idxs[j]])
                    out_buf[pl.ds(obase + j * L, L)] = v

    start_in(0, 0)
    start_in(1, 1)

    def pair_body(k, acc):
        for b in range(2):
            c = 2 * k + b
            wait_in(b)

            @pl.when(c >= 2)
            def _():
                wait_out(b)

            gather_chunk(b)

            @pl.when(c + 2 < N_CHUNKS)
            def _():
                start_in(c + 2, b)

            start_out(c, b)
        return acc

    lax.fori_loop(0, N_CHUNKS // 2, pair_body, 0)
    for b in range(2):
        wait_out(b)


def kernel(x, indices):
    return _permute_sc(x, indices)
